# baseline (speedup 1.0000x reference)
import jax
import jax.numpy as jnp
from jax import lax
from jax.experimental import pallas as pl
from jax.experimental.pallas import tpu as pltpu
from jax.experimental.pallas import tpu_sc as plsc

N = 10000
E = 320000
D = 128
G = 64

NC = 2
NS = 16
NW = NC * NS
CHUNK = 16
EPW = E // NW
NCHUNK = -(-EPW // CHUNK)
EPW_PAD = NCHUNK * CHUNK
NPAD = 10240
ROWS_PER_TILE = NPAD // NS


def _sc_agg_body(h_hbm, src_hbm, dst_hbm, zeros_hbm, out_hbm,
                 sidx_v, didx_v, rows_v, agg_s, sem):
    cid = lax.axis_index("c")
    tid = lax.axis_index("s")
    wid = cid * NS + tid

    zbase = tid * ROWS_PER_TILE
    pltpu.sync_copy(zeros_hbm.at[pl.ds(zbase, ROWS_PER_TILE)],
                    agg_s.at[pl.ds(zbase, ROWS_PER_TILE)])
    plsc.subcore_barrier()

    def body(j, _):
        pltpu.sync_copy(src_hbm.at[wid, j], sidx_v)
        pltpu.sync_copy(dst_hbm.at[wid, j], didx_v)
        pltpu.async_copy(h_hbm.at[sidx_v], rows_v, sem).wait()
        pltpu.sync_copy(rows_v, agg_s.at[didx_v], add=True)
        return 0

    lax.fori_loop(0, NCHUNK, body, 0)
    plsc.subcore_barrier()

    pltpu.sync_copy(agg_s.at[pl.ds(zbase, ROWS_PER_TILE)],
                    out_hbm.at[cid, pl.ds(zbase, ROWS_PER_TILE)])


@jax.jit
def _sc_agg(h, src3, dst3, zeros):
    mesh = plsc.VectorSubcoreMesh(core_axis_name="c", subcore_axis_name="s")
    return pl.kernel(
        _sc_agg_body,
        out_type=jax.ShapeDtypeStruct((NC, NPAD, D), jnp.float32),
        mesh=mesh,
        scratch_types=[
            pltpu.VMEM((CHUNK,), jnp.int32),
            pltpu.VMEM((CHUNK,), jnp.int32),
            pltpu.VMEM((CHUNK, D), jnp.float32),
            pltpu.VMEM_SHARED((NPAD, D), jnp.float32),
            pltpu.SemaphoreType.DMA,
        ],
    )(h, src3, dst3, zeros)


def _dot6(a, b):
    bf = jnp.bfloat16
    f32 = jnp.float32
    a0 = a.astype(bf)
    ra = a - a0.astype(f32)
    a1 = ra.astype(bf)
    a2 = (ra - a1.astype(f32)).astype(bf)
    b0 = b.astype(bf)
    rb = b - b0.astype(f32)
    b1 = rb.astype(bf)
    b2 = (rb - b1.astype(f32)).astype(bf)
    d = lambda x, y: jnp.dot(x, y, preferred_element_type=f32)
    return ((d(a2, b0) + d(a1, b1) + d(a0, b2))
            + (d(a1, b0) + d(a0, b1)) + d(a0, b0))


def _mlp_body(h_ref, agg_ref, eps_ref, w1_ref, w2_ref, vecs_ref, out_ref):
    h = h_ref[...]
    agg = agg_ref[0, :N] + agg_ref[1, :N]
    b1 = vecs_ref[0:1, :]
    g1 = vecs_ref[1:2, :]
    be1 = vecs_ref[2:3, :]
    b2 = vecs_ref[3:4, :]
    go = vecs_ref[4:5, :]
    bo = vecs_ref[5:6, :]

    z = (1.0 + eps_ref[0, 0]) * h + agg
    z = jnp.dot(z, w1_ref[...], preferred_element_type=jnp.float32) + b1
    mu = jnp.mean(z, axis=0, keepdims=True)
    var = jnp.mean((z - mu) * (z - mu), axis=0, keepdims=True)
    z = (z - mu) / jnp.sqrt(var + 1e-5) * g1 + be1
    z = jnp.maximum(z, 0.0)
    z = jnp.dot(z, w2_ref[...], preferred_element_type=jnp.float32) + b2
    mu2 = jnp.mean(z, axis=0, keepdims=True)
    var2 = jnp.mean((z - mu2) * (z - mu2), axis=0, keepdims=True)
    z = (z - mu2) / jnp.sqrt(var2 + 1e-5) * go + bo
    out_ref[...] = jnp.maximum(z, 0.0)


@jax.jit
def _mlp(h, agg, eps_i, w1, w2, vecs):
    return pl.pallas_call(
        _mlp_body,
        out_shape=jax.ShapeDtypeStruct((N, D), jnp.float32),
        in_specs=[
            pl.BlockSpec(memory_space=pltpu.VMEM),
            pl.BlockSpec(memory_space=pltpu.VMEM),
            pl.BlockSpec(memory_space=pltpu.SMEM),
            pl.BlockSpec(memory_space=pltpu.VMEM),
            pl.BlockSpec(memory_space=pltpu.VMEM),
            pl.BlockSpec(memory_space=pltpu.VMEM),
        ],
        out_specs=pl.BlockSpec(memory_space=pltpu.VMEM),
    )(h, agg, eps_i, w1, w2, vecs)


def _pool_body(h_ref, batch_ref, hw_ref, hb_ref, out_ref):
    h = h_ref[...]
    b = batch_ref[...]
    gids = lax.broadcasted_iota(jnp.int32, (G, N), 0)
    onehot = (b == gids).astype(jnp.float32)
    sums = _dot6(onehot, h)
    counts = jnp.sum(onehot, axis=1)[:, None]
    pooled = sums / jnp.maximum(counts, 1.0)
    out_ref[...] = (jnp.dot(pooled, hw_ref[...],
                            preferred_element_type=jnp.float32)
                    + hb_ref[0, 0])


@jax.jit
def _pool(h, batch2, head_W, head_b):
    return pl.pallas_call(
        _pool_body,
        out_shape=jax.ShapeDtypeStruct((G, 1), jnp.float32),
        in_specs=[
            pl.BlockSpec(memory_space=pltpu.VMEM),
            pl.BlockSpec(memory_space=pltpu.VMEM),
            pl.BlockSpec(memory_space=pltpu.VMEM),
            pl.BlockSpec(memory_space=pltpu.SMEM),
        ],
        out_specs=pl.BlockSpec(memory_space=pltpu.VMEM),
    )(h, batch2, head_W, head_b)


def kernel(x, edge_index, batch, eps, W1, b1, g1, be1, W2, b2, go, bo,
           head_W, head_b):
    src = edge_index[0].reshape(NW, EPW)
    dst = edge_index[1].reshape(NW, EPW)
    pad = EPW_PAD - EPW
    if pad:
        src = jnp.concatenate([src, jnp.zeros((NW, pad), jnp.int32)], axis=1)
        dst = jnp.concatenate([dst, jnp.full((NW, pad), N, jnp.int32)], axis=1)
    src3 = src.reshape(NW, NCHUNK, CHUNK)
    dst3 = dst.reshape(NW, NCHUNK, CHUNK)
    zeros = jnp.zeros((NPAD, D), jnp.float32)

    h = x
    for i in range(3):
        agg = _sc_agg(h, src3, dst3, zeros)
        vecs = jnp.stack([b1[i], g1[i], be1[i], b2[i], go[i], bo[i]], axis=0)
        h = _mlp(h, agg, eps[i].reshape(1, 1), W1[i], W2[i], vecs)
    return _pool(h, batch.reshape(1, N), head_W, head_b.reshape(1, 1))


# speedup vs baseline: 2.4681x; 2.4681x over previous
import jax
import jax.numpy as jnp
from jax import lax
from jax.experimental import pallas as pl
from jax.experimental.pallas import tpu as pltpu
from jax.experimental.pallas import tpu_sc as plsc

N = 10000
E = 320000
D = 128
G = 64

NC = 2
NS = 16
NW = NC * NS
CHUNK = 128
EPW = E // NW
NCHUNK = 2 * (-(-EPW // (2 * CHUNK)))
EPW_PAD = NCHUNK * CHUNK
NPAD = 10240
ROWS_PER_TILE = NPAD // NS


def _sc_agg_body(h_hbm, src_hbm, dst_hbm, zeros_hbm, out_hbm,
                 sidx_a, sidx_b, didx_v, rows_a, rows_b, agg_s,
                 sem_a, sem_b):
    cid = lax.axis_index("c")
    tid = lax.axis_index("s")
    wid = cid * NS + tid

    zbase = tid * ROWS_PER_TILE
    pltpu.sync_copy(zeros_hbm.at[pl.ds(zbase, ROWS_PER_TILE)],
                    agg_s.at[pl.ds(zbase, ROWS_PER_TILE)])
    plsc.subcore_barrier()

    npairs = NCHUNK // 2

    pltpu.sync_copy(src_hbm.at[wid, 0], sidx_a)
    pltpu.async_copy(h_hbm.at[sidx_a], rows_a, sem_a)

    def body(k, _):
        ca = 2 * k
        cb = 2 * k + 1
        pltpu.sync_copy(src_hbm.at[wid, cb], sidx_b)
        pltpu.async_copy(h_hbm.at[sidx_b], rows_b, sem_b)
        pltpu.sync_copy(dst_hbm.at[wid, ca], didx_v)
        pltpu.make_async_copy(h_hbm.at[sidx_a], rows_a, sem_a).wait()
        pltpu.sync_copy(rows_a, agg_s.at[didx_v], add=True)

        @pl.when(k < npairs - 1)
        def _():
            pltpu.sync_copy(src_hbm.at[wid, ca + 2], sidx_a)
            pltpu.async_copy(h_hbm.at[sidx_a], rows_a, sem_a)

        pltpu.sync_copy(dst_hbm.at[wid, cb], didx_v)
        pltpu.make_async_copy(h_hbm.at[sidx_b], rows_b, sem_b).wait()
        pltpu.sync_copy(rows_b, agg_s.at[didx_v], add=True)
        return 0

    lax.fori_loop(0, npairs, body, 0)
    plsc.subcore_barrier()

    pltpu.sync_copy(agg_s.at[pl.ds(zbase, ROWS_PER_TILE)],
                    out_hbm.at[cid, pl.ds(zbase, ROWS_PER_TILE)])


@jax.jit
def _sc_agg(h, src3, dst3, zeros):
    mesh = plsc.VectorSubcoreMesh(core_axis_name="c", subcore_axis_name="s")
    return pl.kernel(
        _sc_agg_body,
        out_type=jax.ShapeDtypeStruct((NC, NPAD, D), jnp.float32),
        mesh=mesh,
        scratch_types=[
            pltpu.VMEM((CHUNK,), jnp.int32),
            pltpu.VMEM((CHUNK,), jnp.int32),
            pltpu.VMEM((CHUNK,), jnp.int32),
            pltpu.VMEM((CHUNK, D), jnp.float32),
            pltpu.VMEM((CHUNK, D), jnp.float32),
            pltpu.VMEM_SHARED((NPAD, D), jnp.float32),
            pltpu.SemaphoreType.DMA,
            pltpu.SemaphoreType.DMA,
        ],
    )(h, src3, dst3, zeros)


def _dot6(a, b):
    bf = jnp.bfloat16
    f32 = jnp.float32
    a0 = a.astype(bf)
    ra = a - a0.astype(f32)
    a1 = ra.astype(bf)
    a2 = (ra - a1.astype(f32)).astype(bf)
    b0 = b.astype(bf)
    rb = b - b0.astype(f32)
    b1 = rb.astype(bf)
    b2 = (rb - b1.astype(f32)).astype(bf)
    d = lambda x, y: jnp.dot(x, y, preferred_element_type=f32)
    return ((d(a2, b0) + d(a1, b1) + d(a0, b2))
            + (d(a1, b0) + d(a0, b1)) + d(a0, b0))


def _mlp_body(h_ref, agg_ref, eps_ref, w1_ref, w2_ref, vecs_ref, out_ref):
    h = h_ref[...]
    agg = agg_ref[0, :N] + agg_ref[1, :N]
    b1 = vecs_ref[0:1, :]
    g1 = vecs_ref[1:2, :]
    be1 = vecs_ref[2:3, :]
    b2 = vecs_ref[3:4, :]
    go = vecs_ref[4:5, :]
    bo = vecs_ref[5:6, :]

    z = (1.0 + eps_ref[0, 0]) * h + agg
    z = jnp.dot(z, w1_ref[...], preferred_element_type=jnp.float32) + b1
    mu = jnp.mean(z, axis=0, keepdims=True)
    var = jnp.mean((z - mu) * (z - mu), axis=0, keepdims=True)
    z = (z - mu) / jnp.sqrt(var + 1e-5) * g1 + be1
    z = jnp.maximum(z, 0.0)
    z = jnp.dot(z, w2_ref[...], preferred_element_type=jnp.float32) + b2
    mu2 = jnp.mean(z, axis=0, keepdims=True)
    var2 = jnp.mean((z - mu2) * (z - mu2), axis=0, keepdims=True)
    z = (z - mu2) / jnp.sqrt(var2 + 1e-5) * go + bo
    out_ref[...] = jnp.maximum(z, 0.0)


@jax.jit
def _mlp(h, agg, eps_i, w1, w2, vecs):
    return pl.pallas_call(
        _mlp_body,
        out_shape=jax.ShapeDtypeStruct((N, D), jnp.float32),
        in_specs=[
            pl.BlockSpec(memory_space=pltpu.VMEM),
            pl.BlockSpec(memory_space=pltpu.VMEM),
            pl.BlockSpec(memory_space=pltpu.SMEM),
            pl.BlockSpec(memory_space=pltpu.VMEM),
            pl.BlockSpec(memory_space=pltpu.VMEM),
            pl.BlockSpec(memory_space=pltpu.VMEM),
        ],
        out_specs=pl.BlockSpec(memory_space=pltpu.VMEM),
    )(h, agg, eps_i, w1, w2, vecs)


def _pool_body(h_ref, batch_ref, hw_ref, hb_ref, out_ref):
    h = h_ref[...]
    b = batch_ref[...]
    gids = lax.broadcasted_iota(jnp.int32, (G, N), 0)
    onehot = (b == gids).astype(jnp.float32)
    sums = _dot6(onehot, h)
    counts = jnp.sum(onehot, axis=1)[:, None]
    pooled = sums / jnp.maximum(counts, 1.0)
    out_ref[...] = (jnp.dot(pooled, hw_ref[...],
                            preferred_element_type=jnp.float32)
                    + hb_ref[0, 0])


@jax.jit
def _pool(h, batch2, head_W, head_b):
    return pl.pallas_call(
        _pool_body,
        out_shape=jax.ShapeDtypeStruct((G, 1), jnp.float32),
        in_specs=[
            pl.BlockSpec(memory_space=pltpu.VMEM),
            pl.BlockSpec(memory_space=pltpu.VMEM),
            pl.BlockSpec(memory_space=pltpu.VMEM),
            pl.BlockSpec(memory_space=pltpu.SMEM),
        ],
        out_specs=pl.BlockSpec(memory_space=pltpu.VMEM),
    )(h, batch2, head_W, head_b)


def kernel(x, edge_index, batch, eps, W1, b1, g1, be1, W2, b2, go, bo,
           head_W, head_b):
    src = edge_index[0].reshape(NW, EPW)
    dst = edge_index[1].reshape(NW, EPW)
    pad = EPW_PAD - EPW
    if pad:
        src = jnp.concatenate([src, jnp.zeros((NW, pad), jnp.int32)], axis=1)
        dst = jnp.concatenate([dst, jnp.full((NW, pad), N, jnp.int32)], axis=1)
    src3 = src.reshape(NW, NCHUNK, CHUNK)
    dst3 = dst.reshape(NW, NCHUNK, CHUNK)
    zeros = jnp.zeros((NPAD, D), jnp.float32)

    h = x
    for i in range(3):
        agg = _sc_agg(h, src3, dst3, zeros)
        vecs = jnp.stack([b1[i], g1[i], be1[i], b2[i], go[i], bo[i]], axis=0)
        h = _mlp(h, agg, eps[i].reshape(1, 1), W1[i], W2[i], vecs)
    return _pool(h, batch.reshape(1, N), head_W, head_b.reshape(1, 1))


# speedup vs baseline: 2.8491x; 1.1543x over previous
import jax
import jax.numpy as jnp
from jax import lax
from jax.experimental import pallas as pl
from jax.experimental.pallas import tpu as pltpu
from jax.experimental.pallas import tpu_sc as plsc

N = 10000
E = 320000
D = 128
G = 64

NC = 2
NS = 16
NW = NC * NS
CHUNK = 128
EPW = E // NW
NCHUNK = -(-EPW // CHUNK)
EPW_PAD = NCHUNK * CHUNK
NPAD = 10240
ROWS_PER_TILE = NPAD // NS


def _sc_agg_body(h_hbm, src_hbm, dst_hbm, zeros_hbm, out_hbm,
                 sidx_v, didx_v, rows_v, agg_s, sem):
    cid = lax.axis_index("c")
    tid = lax.axis_index("s")
    wid = cid * NS + tid

    zbase = tid * ROWS_PER_TILE
    pltpu.sync_copy(zeros_hbm.at[pl.ds(zbase, ROWS_PER_TILE)],
                    agg_s.at[pl.ds(zbase, ROWS_PER_TILE)])
    plsc.subcore_barrier()

    def body(j, _):
        pltpu.sync_copy(src_hbm.at[wid, j], sidx_v)
        pltpu.sync_copy(dst_hbm.at[wid, j], didx_v)
        pltpu.async_copy(h_hbm.at[sidx_v], rows_v, sem).wait()
        pltpu.sync_copy(rows_v, agg_s.at[didx_v], add=True)
        return 0

    lax.fori_loop(0, NCHUNK, body, 0)
    plsc.subcore_barrier()

    pltpu.sync_copy(agg_s.at[pl.ds(zbase, ROWS_PER_TILE)],
                    out_hbm.at[cid, pl.ds(zbase, ROWS_PER_TILE)])


@jax.jit
def _sc_agg(h, src3, dst3, zeros):
    mesh = plsc.VectorSubcoreMesh(core_axis_name="c", subcore_axis_name="s")
    return pl.kernel(
        _sc_agg_body,
        out_type=jax.ShapeDtypeStruct((NC, NPAD, D), jnp.float32),
        mesh=mesh,
        scratch_types=[
            pltpu.VMEM((CHUNK,), jnp.int32),
            pltpu.VMEM((CHUNK,), jnp.int32),
            pltpu.VMEM((CHUNK, D), jnp.float32),
            pltpu.VMEM_SHARED((NPAD, D), jnp.float32),
            pltpu.SemaphoreType.DMA,
        ],
    )(h, src3, dst3, zeros)


def _dot6(a, b):
    bf = jnp.bfloat16
    f32 = jnp.float32
    a0 = a.astype(bf)
    ra = a - a0.astype(f32)
    a1 = ra.astype(bf)
    a2 = (ra - a1.astype(f32)).astype(bf)
    b0 = b.astype(bf)
    rb = b - b0.astype(f32)
    b1 = rb.astype(bf)
    b2 = (rb - b1.astype(f32)).astype(bf)
    d = lambda x, y: jnp.dot(x, y, preferred_element_type=f32)
    return ((d(a2, b0) + d(a1, b1) + d(a0, b2))
            + (d(a1, b0) + d(a0, b1)) + d(a0, b0))


def _mlp_body(h_ref, agg_ref, eps_ref, w1_ref, w2_ref, vecs_ref, out_ref):
    h = h_ref[...]
    agg = agg_ref[0, :N] + agg_ref[1, :N]
    b1 = vecs_ref[0:1, :]
    g1 = vecs_ref[1:2, :]
    be1 = vecs_ref[2:3, :]
    b2 = vecs_ref[3:4, :]
    go = vecs_ref[4:5, :]
    bo = vecs_ref[5:6, :]

    z = (1.0 + eps_ref[0, 0]) * h + agg
    z = jnp.dot(z, w1_ref[...], preferred_element_type=jnp.float32) + b1
    mu = jnp.mean(z, axis=0, keepdims=True)
    var = jnp.mean((z - mu) * (z - mu), axis=0, keepdims=True)
    z = (z - mu) / jnp.sqrt(var + 1e-5) * g1 + be1
    z = jnp.maximum(z, 0.0)
    z = jnp.dot(z, w2_ref[...], preferred_element_type=jnp.float32) + b2
    mu2 = jnp.mean(z, axis=0, keepdims=True)
    var2 = jnp.mean((z - mu2) * (z - mu2), axis=0, keepdims=True)
    z = (z - mu2) / jnp.sqrt(var2 + 1e-5) * go + bo
    out_ref[...] = jnp.maximum(z, 0.0)


@jax.jit
def _mlp(h, agg, eps_i, w1, w2, vecs):
    return pl.pallas_call(
        _mlp_body,
        out_shape=jax.ShapeDtypeStruct((N, D), jnp.float32),
        in_specs=[
            pl.BlockSpec(memory_space=pltpu.VMEM),
            pl.BlockSpec(memory_space=pltpu.VMEM),
            pl.BlockSpec(memory_space=pltpu.SMEM),
            pl.BlockSpec(memory_space=pltpu.VMEM),
            pl.BlockSpec(memory_space=pltpu.VMEM),
            pl.BlockSpec(memory_space=pltpu.VMEM),
        ],
        out_specs=pl.BlockSpec(memory_space=pltpu.VMEM),
    )(h, agg, eps_i, w1, w2, vecs)


def _pool_body(h_ref, batch_ref, hw_ref, hb_ref, out_ref):
    h = h_ref[...]
    b = batch_ref[...]
    gids = lax.broadcasted_iota(jnp.int32, (G, N), 0)
    onehot = (b == gids).astype(jnp.float32)
    sums = _dot6(onehot, h)
    counts = jnp.sum(onehot, axis=1)[:, None]
    pooled = sums / jnp.maximum(counts, 1.0)
    out_ref[...] = (jnp.dot(pooled, hw_ref[...],
                            preferred_element_type=jnp.float32)
                    + hb_ref[0, 0])


@jax.jit
def _pool(h, batch2, head_W, head_b):
    return pl.pallas_call(
        _pool_body,
        out_shape=jax.ShapeDtypeStruct((G, 1), jnp.float32),
        in_specs=[
            pl.BlockSpec(memory_space=pltpu.VMEM),
            pl.BlockSpec(memory_space=pltpu.VMEM),
            pl.BlockSpec(memory_space=pltpu.VMEM),
            pl.BlockSpec(memory_space=pltpu.SMEM),
        ],
        out_specs=pl.BlockSpec(memory_space=pltpu.VMEM),
    )(h, batch2, head_W, head_b)


def kernel(x, edge_index, batch, eps, W1, b1, g1, be1, W2, b2, go, bo,
           head_W, head_b):
    src = edge_index[0].reshape(NW, EPW)
    dst = edge_index[1].reshape(NW, EPW)
    pad = EPW_PAD - EPW
    if pad:
        src = jnp.concatenate([src, jnp.zeros((NW, pad), jnp.int32)], axis=1)
        dst = jnp.concatenate([dst, jnp.full((NW, pad), N, jnp.int32)], axis=1)
    src3 = src.reshape(NW, NCHUNK, CHUNK)
    dst3 = dst.reshape(NW, NCHUNK, CHUNK)
    zeros = jnp.zeros((NPAD, D), jnp.float32)

    h = x
    for i in range(3):
        agg = _sc_agg(h, src3, dst3, zeros)
        vecs = jnp.stack([b1[i], g1[i], be1[i], b2[i], go[i], bo[i]], axis=0)
        h = _mlp(h, agg, eps[i].reshape(1, 1), W1[i], W2[i], vecs)
    return _pool(h, batch.reshape(1, N), head_W, head_b.reshape(1, 1))
